# baseline (speedup 1.0000x reference)
import functools

import jax
import jax.numpy as jnp
from jax.experimental import pallas as pl
from jax.experimental.pallas import tpu as pltpu

_EPS = 1e-5
_BF16 = jnp.bfloat16
_F32 = jnp.float32


def _make_stage_kernels(H, W, C, d):
    Hq = H + 2 * d
    L = 16
    Wq = 64
    R = d
    M = H * W

    def _fill_padded(hp_ref, interior_bf16):
        z = jnp.zeros((R, Wq, C), _BF16)
        hp_ref[0:R] = z
        hp_ref[R + H:Hq] = z
        hp_ref[R:R + H, 0:L] = jnp.zeros((H, L, C), _BF16)
        hp_ref[R:R + H, L + W:Wq] = jnp.zeros((H, Wq - L - W, C), _BF16)
        hp_ref[R:R + H, L:L + W] = interior_bf16.reshape(H, W, C)

    def _patch(hp_ref):
        taps = [hp_ref[kh * d:kh * d + H,
                       L - d + kw * d:L - d + kw * d + W, :].reshape(M, C)
                for kh in range(3) for kw in range(3)]
        return jnp.concatenate(taps, axis=1)

    def stage1(x_ref, eye_ref, w_ref, y_ref, st_ref, xb_ref, hp_ref):
        for b in range(x_ref.shape[0]):
            xb = x_ref[b].astype(_BF16)
            xb_ref[b] = xb
            xt = jax.lax.dot_general(
                xb, eye_ref[...], (((0,), (0,)), ((), ())),
                preferred_element_type=_F32)
            hp = hp_ref.at[b]
            _fill_padded(hp, xt.astype(_BF16))
            y = jnp.dot(_patch(hp), w_ref[...],
                        preferred_element_type=_F32)
            st_ref[b] = jnp.stack([jnp.sum(y, axis=0),
                                   jnp.sum(y * y, axis=0)])
            y_ref[b] = y.astype(_BF16)

    def _affine(st_ref, g_ref, be_ref, count):
        s = jnp.sum(st_ref[...], axis=0)
        mean = s[0:1] / count
        var = s[1:2] / count - mean * mean
        sc = g_ref[...] * jax.lax.rsqrt(var + _EPS)
        sh = be_ref[...] - mean * sc
        return sc, sh

    def stage2(y1_ref, st_ref, g_ref, be_ref, w_ref, y_ref, sto_ref, hp_ref,
               *, count):
        sc, sh = _affine(st_ref, g_ref, be_ref, count)
        for b in range(y1_ref.shape[0]):
            a = y1_ref[b].astype(_F32) * sc + sh
            hp = hp_ref.at[b]
            _fill_padded(hp, jnp.maximum(a, 0.0).astype(_BF16))
            y = jnp.dot(_patch(hp), w_ref[...],
                        preferred_element_type=_F32)
            sto_ref[b] = jnp.stack([jnp.sum(y, axis=0),
                                    jnp.sum(y * y, axis=0)])
            y_ref[b] = y.astype(_BF16)

    def stage3(y2_ref, st_ref, g_ref, be_ref, eye_ref, x_ref, o_ref,
               *, count):
        sc, sh = _affine(st_ref, g_ref, be_ref, count)
        sct = jnp.transpose(sc)
        sht = jnp.transpose(sh)
        for b in range(y2_ref.shape[0]):
            zt = jax.lax.dot_general(
                eye_ref[...], y2_ref[b], (((1,), (1,)), ((), ())),
                preferred_element_type=_F32)
            o_ref[b] = zt * sct + sht + x_ref[b].astype(_F32)

    return stage1, stage2, stage3


@jax.jit
def _forward(x, w1, g1, be1, w2, g2, be2):
    N, C, H, W = x.shape
    d = 2
    Hp, Wp = H + 2 * d, W + 2 * d
    M = H * W
    count = float(N * H * W)
    B12 = 4 if N % 4 == 0 else 1
    B3 = 4 if N % 4 == 0 else 1

    w1s = jnp.transpose(w1, (2, 3, 1, 0)).reshape(9 * C, C).astype(_BF16)
    w2s = jnp.transpose(w2, (2, 3, 1, 0)).reshape(9 * C, C).astype(_BF16)
    eye_c = jnp.eye(C, dtype=_BF16)

    xm = x.reshape(N, C, M)

    k1, k2, k3 = _make_stage_kernels(H, W, C, d)
    cparams = pltpu.CompilerParams(
        dimension_semantics=("parallel",),
        vmem_limit_bytes=64 * 1024 * 1024,
    )

    y1, st1, xbm = pl.pallas_call(
        k1,
        grid=(N // B12,),
        in_specs=[
            pl.BlockSpec((B12, C, M), lambda n: (n, 0, 0)),
            pl.BlockSpec((C, C), lambda n: (0, 0)),
            pl.BlockSpec((9 * C, C), lambda n: (0, 0)),
        ],
        out_specs=(
            pl.BlockSpec((B12, M, C), lambda n: (n, 0, 0)),
            pl.BlockSpec((B12, 2, C), lambda n: (n, 0, 0)),
            pl.BlockSpec((B12, C, M), lambda n: (n, 0, 0)),
        ),
        out_shape=(
            jax.ShapeDtypeStruct((N, M, C), _BF16),
            jax.ShapeDtypeStruct((N, 2, C), _F32),
            jax.ShapeDtypeStruct((N, C, M), _BF16),
        ),
        scratch_shapes=[pltpu.VMEM((B12, H + 2 * d, 64, C), _BF16)],
        compiler_params=cparams,
    )(xm, eye_c, w1s)

    g1r, be1r = g1.reshape(1, C).astype(_F32), be1.reshape(1, C).astype(_F32)
    g2r, be2r = g2.reshape(1, C).astype(_F32), be2.reshape(1, C).astype(_F32)

    y2, st2 = pl.pallas_call(
        functools.partial(k2, count=count),
        grid=(N // B12,),
        in_specs=[
            pl.BlockSpec((B12, M, C), lambda n: (n, 0, 0)),
            pl.BlockSpec((N, 2, C), lambda n: (0, 0, 0)),
            pl.BlockSpec((1, C), lambda n: (0, 0)),
            pl.BlockSpec((1, C), lambda n: (0, 0)),
            pl.BlockSpec((9 * C, C), lambda n: (0, 0)),
        ],
        out_specs=(
            pl.BlockSpec((B12, M, C), lambda n: (n, 0, 0)),
            pl.BlockSpec((B12, 2, C), lambda n: (n, 0, 0)),
        ),
        out_shape=(
            jax.ShapeDtypeStruct((N, M, C), _BF16),
            jax.ShapeDtypeStruct((N, 2, C), _F32),
        ),
        scratch_shapes=[pltpu.VMEM((B12, H + 2 * d, 64, C), _BF16)],
        compiler_params=cparams,
    )(y1, st1, g1r, be1r, w2s)

    out = pl.pallas_call(
        functools.partial(k3, count=count),
        grid=(N // B3,),
        in_specs=[
            pl.BlockSpec((B3, M, C), lambda n: (n, 0, 0)),
            pl.BlockSpec((N, 2, C), lambda n: (0, 0, 0)),
            pl.BlockSpec((1, C), lambda n: (0, 0)),
            pl.BlockSpec((1, C), lambda n: (0, 0)),
            pl.BlockSpec((C, C), lambda n: (0, 0)),
            pl.BlockSpec((B3, C, M), lambda n: (n, 0, 0)),
        ],
        out_specs=pl.BlockSpec((B3, C, M), lambda n: (n, 0, 0)),
        out_shape=jax.ShapeDtypeStruct((N, C, M), _F32),
        compiler_params=cparams,
    )(y2, st2, g2r, be2r, eye_c, xbm)
    return out.reshape(N, C, H, W)


def kernel(x, w1, b1, g1, be1, w2, b2, g2, be2):
    del b1, b2
    return _forward(x, w1, g1, be1, w2, g2, be2)


# speedup vs baseline: 1.0061x; 1.0061x over previous
import functools

import jax
import jax.numpy as jnp
from jax.experimental import pallas as pl
from jax.experimental.pallas import tpu as pltpu

_EPS = 1e-5
_BF16 = jnp.bfloat16
_F32 = jnp.float32


def _make_stage_kernels(H, W, C, d):
    Hq = H + 2 * d
    L = 16
    Wq = 64
    R = d
    M = H * W

    def _fill_padded(hp_ref, interior_bf16):
        z = jnp.zeros((R, Wq, C), _BF16)
        hp_ref[0:R] = z
        hp_ref[R + H:Hq] = z
        hp_ref[R:R + H, 0:L] = jnp.zeros((H, L, C), _BF16)
        hp_ref[R:R + H, L + W:Wq] = jnp.zeros((H, Wq - L - W, C), _BF16)
        hp_ref[R:R + H, L:L + W] = interior_bf16.reshape(H, W, C)

    def _patch(hp_ref):
        taps = [hp_ref[kh * d:kh * d + H,
                       L - d + kw * d:L - d + kw * d + W, :].reshape(M, C)
                for kh in range(3) for kw in range(3)]
        return jnp.concatenate(taps, axis=1)

    def stage1(x_ref, eye_ref, w_ref, y_ref, st_ref, xb_ref, hp_ref):
        for b in range(x_ref.shape[0]):
            xb = x_ref[b].astype(_BF16)
            xb_ref[b] = xb
            xt = jax.lax.dot_general(
                xb, eye_ref[...], (((0,), (0,)), ((), ())),
                preferred_element_type=_F32)
            hp = hp_ref.at[b]
            _fill_padded(hp, xt.astype(_BF16))
            y = jnp.dot(_patch(hp), w_ref[...],
                        preferred_element_type=_F32)
            st_ref[b] = jnp.stack([jnp.sum(y, axis=0),
                                   jnp.sum(y * y, axis=0)])
            y_ref[b] = y.astype(_BF16)

    def _affine(st_ref, g_ref, be_ref, count):
        s = jnp.sum(st_ref[...], axis=0)
        mean = s[0:1] / count
        var = s[1:2] / count - mean * mean
        sc = g_ref[...] * jax.lax.rsqrt(var + _EPS)
        sh = be_ref[...] - mean * sc
        return sc, sh

    def stage2(y1_ref, st_ref, g_ref, be_ref, w_ref, y_ref, sto_ref, hp_ref,
               *, count):
        sc, sh = _affine(st_ref, g_ref, be_ref, count)
        for b in range(y1_ref.shape[0]):
            a = y1_ref[b].astype(_F32) * sc + sh
            hp = hp_ref.at[b]
            _fill_padded(hp, jnp.maximum(a, 0.0).astype(_BF16))
            y = jnp.dot(_patch(hp), w_ref[...],
                        preferred_element_type=_F32)
            sto_ref[b] = jnp.stack([jnp.sum(y, axis=0),
                                    jnp.sum(y * y, axis=0)])
            y_ref[b] = y.astype(_BF16)

    def stage3(y2_ref, st_ref, g_ref, be_ref, eye_ref, x_ref, o_ref,
               *, count):
        sc, sh = _affine(st_ref, g_ref, be_ref, count)
        sct = jnp.transpose(sc)
        sht = jnp.transpose(sh)
        for b in range(y2_ref.shape[0]):
            zt = jax.lax.dot_general(
                eye_ref[...], y2_ref[b], (((1,), (1,)), ((), ())),
                preferred_element_type=_F32)
            o_ref[b] = zt * sct + sht + x_ref[b].astype(_F32)

    return stage1, stage2, stage3


@jax.jit
def _forward(x, w1, g1, be1, w2, g2, be2):
    N, C, H, W = x.shape
    d = 2
    Hp, Wp = H + 2 * d, W + 2 * d
    M = H * W
    count = float(N * H * W)
    B12 = 4 if N % 4 == 0 else 1
    B3 = 8 if N % 8 == 0 else (4 if N % 4 == 0 else 1)

    w1s = jnp.transpose(w1, (2, 3, 1, 0)).reshape(9 * C, C).astype(_BF16)
    w2s = jnp.transpose(w2, (2, 3, 1, 0)).reshape(9 * C, C).astype(_BF16)
    eye_c = jnp.eye(C, dtype=_BF16)

    xm = x.reshape(N, C, M)

    k1, k2, k3 = _make_stage_kernels(H, W, C, d)
    cparams = pltpu.CompilerParams(
        dimension_semantics=("parallel",),
        vmem_limit_bytes=64 * 1024 * 1024,
    )

    y1, st1, xbm = pl.pallas_call(
        k1,
        grid=(N // B12,),
        in_specs=[
            pl.BlockSpec((B12, C, M), lambda n: (n, 0, 0)),
            pl.BlockSpec((C, C), lambda n: (0, 0)),
            pl.BlockSpec((9 * C, C), lambda n: (0, 0)),
        ],
        out_specs=(
            pl.BlockSpec((B12, M, C), lambda n: (n, 0, 0)),
            pl.BlockSpec((B12, 2, C), lambda n: (n, 0, 0)),
            pl.BlockSpec((B12, C, M), lambda n: (n, 0, 0)),
        ),
        out_shape=(
            jax.ShapeDtypeStruct((N, M, C), _BF16),
            jax.ShapeDtypeStruct((N, 2, C), _F32),
            jax.ShapeDtypeStruct((N, C, M), _BF16),
        ),
        scratch_shapes=[pltpu.VMEM((B12, H + 2 * d, 64, C), _BF16)],
        compiler_params=cparams,
    )(xm, eye_c, w1s)

    g1r, be1r = g1.reshape(1, C).astype(_F32), be1.reshape(1, C).astype(_F32)
    g2r, be2r = g2.reshape(1, C).astype(_F32), be2.reshape(1, C).astype(_F32)

    y2, st2 = pl.pallas_call(
        functools.partial(k2, count=count),
        grid=(N // B12,),
        in_specs=[
            pl.BlockSpec((B12, M, C), lambda n: (n, 0, 0)),
            pl.BlockSpec((N, 2, C), lambda n: (0, 0, 0)),
            pl.BlockSpec((1, C), lambda n: (0, 0)),
            pl.BlockSpec((1, C), lambda n: (0, 0)),
            pl.BlockSpec((9 * C, C), lambda n: (0, 0)),
        ],
        out_specs=(
            pl.BlockSpec((B12, M, C), lambda n: (n, 0, 0)),
            pl.BlockSpec((B12, 2, C), lambda n: (n, 0, 0)),
        ),
        out_shape=(
            jax.ShapeDtypeStruct((N, M, C), _BF16),
            jax.ShapeDtypeStruct((N, 2, C), _F32),
        ),
        scratch_shapes=[pltpu.VMEM((B12, H + 2 * d, 64, C), _BF16)],
        compiler_params=cparams,
    )(y1, st1, g1r, be1r, w2s)

    out = pl.pallas_call(
        functools.partial(k3, count=count),
        grid=(N // B3,),
        in_specs=[
            pl.BlockSpec((B3, M, C), lambda n: (n, 0, 0)),
            pl.BlockSpec((N, 2, C), lambda n: (0, 0, 0)),
            pl.BlockSpec((1, C), lambda n: (0, 0)),
            pl.BlockSpec((1, C), lambda n: (0, 0)),
            pl.BlockSpec((C, C), lambda n: (0, 0)),
            pl.BlockSpec((B3, C, M), lambda n: (n, 0, 0)),
        ],
        out_specs=pl.BlockSpec((B3, C, M), lambda n: (n, 0, 0)),
        out_shape=jax.ShapeDtypeStruct((N, C, M), _F32),
        compiler_params=cparams,
    )(y2, st2, g2r, be2r, eye_c, xbm)
    return out.reshape(N, C, H, W)


def kernel(x, w1, b1, g1, be1, w2, b2, g2, be2):
    del b1, b2
    return _forward(x, w1, g1, be1, w2, g2, be2)
